# baseline (speedup 1.0000x reference)
import functools

import jax
import jax.numpy as jnp
from jax import lax
from jax.experimental import pallas as pl
from jax.experimental.pallas import tpu as pltpu
from jax.experimental.pallas import tpu_sc as plsc

S = 2
FIN = 128
FOUT = 64
NR = 40962
NV = 163842
NEI = 7
ROWS = S * FOUT

NC, NS, L = 2, 16, 16
NW = NC * NS

NRL = 41088
NPL = 8
DPAD = 40968

RPW = 1280
RTAIL = NR - NW * RPW
RPA = 1296

NVP = 163968
HSPLIT = 81920
H1W = 82048
YB = 82048
CHK = 8192
NCH = 40960 // CHK
VUNROLL = 8
PPW = FOUT // NW


def _mesh():
    return plsc.VectorSubcoreMesh(core_axis_name="c", subcore_axis_name="s")


def _wid():
    return lax.axis_index("s") * NC + lax.axis_index("c")


def _neigh_gather(up_flat, down_pad):
    @functools.partial(
        pl.kernel,
        mesh=_mesh(),
        out_type=jax.ShapeDtypeStruct((NPL * NRL,), jnp.int32),
        scratch_types=[
            pltpu.VMEM((128,), jnp.int32),
            pltpu.VMEM((NEI * 128,), jnp.int32),
            pltpu.VMEM((NEI * RPA,), jnp.int32),
            pltpu.VMEM((16,), jnp.int32),
            pltpu.SemaphoreType.DMA,
        ],
    )
    def k(up_hbm, down_hbm, out_hbm, didx, gidx, rt_v, dt, sem):
        wid = _wid()
        base = wid * RPW
        for j in range(RPW // 128):
            pltpu.sync_copy(down_hbm.at[pl.ds(base + j * 128, 128)], didx)

            def cv(i, _):
                d16 = didx[pl.ds(i * L, L)]
                for kk in range(NEI):
                    gidx[pl.ds(kk * 128 + i * L, L)] = d16 * NEI + kk
                return 0

            lax.fori_loop(0, 128 // L, cv, 0)
            for kk in range(NEI):
                pltpu.async_copy(up_hbm.at[gidx.at[pl.ds(kk * 128, 128)]],
                                 rt_v.at[pl.ds(kk * RPA + j * 128, 128)],
                                 sem)
            for kk in range(NEI):
                pltpu.make_async_copy(
                    up_hbm.at[gidx.at[pl.ds(kk * 128, 128)]],
                    rt_v.at[pl.ds(kk * RPA + j * 128, 128)], sem).wait()

        @pl.when(wid == NW - 1)
        def _():
            dt[...] = jnp.zeros((L,), jnp.int32)
            pltpu.sync_copy(down_hbm.at[pl.ds(NW * RPW, 8)],
                            dt.at[pl.ds(0, 8)])
            dv = dt[...]
            for kk in range(NEI):
                gidx[pl.ds(kk * 128, L)] = dv * NEI + kk
            for kk in range(NEI):
                pltpu.async_copy(up_hbm.at[gidx.at[pl.ds(kk * 128, L)]],
                                 rt_v.at[pl.ds(kk * RPA + RPW, L)], sem)
            for kk in range(NEI):
                pltpu.make_async_copy(
                    up_hbm.at[gidx.at[pl.ds(kk * 128, L)]],
                    rt_v.at[pl.ds(kk * RPA + RPW, L)], sem).wait()

        for kk in range(NEI):
            @pl.when(wid < NW - 1)
            def _(kk=kk):
                pltpu.sync_copy(rt_v.at[pl.ds(kk * RPA, RPW)],
                                out_hbm.at[pl.ds(kk * NRL + base, RPW)])

            @pl.when(wid == NW - 1)
            def _(kk=kk):
                pltpu.sync_copy(rt_v.at[pl.ds(kk * RPA, 1288)],
                                out_hbm.at[pl.ds(kk * NRL + base, 1288)])

    return k(up_flat, down_pad)


_NB = 4096


def _fc_body(x_ref, mpi_ref, nt_ref, w_ref, b_ref, h_ref, vi_ref):
    h_ref[0] = (
        jnp.dot(w_ref[...], x_ref[0], preferred_element_type=jnp.float32)
        + b_ref[...]
    )
    mpi = mpi_ref[0]
    nt = nt_ref[...]
    acc = jnp.broadcast_to(nt[NEI - 1][None, :], mpi.shape)
    for kk in range(NEI - 2, -1, -1):
        acc = jnp.where(mpi == kk, nt[kk][None, :], acc)
    vi_ref[0] = acc


def _fc_vi(x, mpi, neigh_t, W, b2):
    grid = (S, pl.cdiv(NR, _NB))
    return pl.pallas_call(
        _fc_body,
        grid=grid,
        in_specs=[
            pl.BlockSpec((1, FIN, _NB), lambda s, j: (s, 0, j)),
            pl.BlockSpec((1, FOUT, _NB), lambda s, j: (s, 0, j)),
            pl.BlockSpec((NPL, _NB), lambda s, j: (0, j)),
            pl.BlockSpec((FOUT, FIN), lambda s, j: (0, 0)),
            pl.BlockSpec((FOUT, _NB), lambda s, j: (0, 0)),
        ],
        out_specs=[
            pl.BlockSpec((1, FOUT, _NB), lambda s, j: (s, 0, j)),
            pl.BlockSpec((1, FOUT, _NB), lambda s, j: (s, 0, j)),
        ],
        out_shape=[
            jax.ShapeDtypeStruct((S, FOUT, NRL), jnp.float32),
            jax.ShapeDtypeStruct((S, FOUT, NRL), jnp.int32),
        ],
    )(x, mpi, neigh_t, W, b2)


def _scatter_rows(vi_flat, h_flat):
    @functools.partial(
        pl.kernel,
        mesh=_mesh(),
        out_type=jax.ShapeDtypeStruct((ROWS * NVP,), jnp.float32),
        compiler_params=pltpu.CompilerParams(needs_layout_passes=False),
        scratch_types=[
            pltpu.VMEM((YB,), jnp.float32),
            pltpu.VMEM((2 * CHK,), jnp.int32),
            pltpu.VMEM((2 * CHK,), jnp.float32),
            pltpu.VMEM((16,), jnp.int32),
            pltpu.VMEM((16,), jnp.float32),
            pltpu.SemaphoreType.DMA,
            pltpu.SemaphoreType.DMA,
        ],
    )
    def k(vi_hbm, h_hbm, y_hbm, ybuf, vib, hb, vtl, htl, sem_vi, sem_h):
        wid = _wid()
        iota = lax.iota(jnp.int32, L)
        zeros = jnp.zeros((L,), jnp.float32)

        def scan_row(row, lo, hi):
            rb = row * NRL

            def fire(c, p):
                pltpu.async_copy(vi_hbm.at[pl.ds(rb + c * CHK, CHK)],
                                 vib.at[pl.ds(p * CHK, CHK)], sem_vi)
                pltpu.async_copy(h_hbm.at[pl.ds(rb + c * CHK, CHK)],
                                 hb.at[pl.ds(p * CHK, CHK)], sem_h)

            def wait(c, p):
                pltpu.make_async_copy(vi_hbm.at[pl.ds(rb + c * CHK, CHK)],
                                      vib.at[pl.ds(p * CHK, CHK)],
                                      sem_vi).wait()
                pltpu.make_async_copy(h_hbm.at[pl.ds(rb + c * CHK, CHK)],
                                      hb.at[pl.ds(p * CHK, CHK)],
                                      sem_h).wait()

            fire(0, 0)

            def chunk(c, _):
                p = lax.rem(c, 2)
                wait(c, p)

                @pl.when(c + 1 < NCH)
                def _():
                    fire(c + 1, 1 - p)

                def vec(i, _):
                    for u in range(VUNROLL):
                        off = p * CHK + i * (L * VUNROLL) + u * L
                        v = vib[pl.ds(off, L)]
                        hv = hb[pl.ds(off, L)]
                        m = (v >= lo) & (v < hi)
                        t = jnp.where(m, v - lo, 0)
                        plsc.store_scatter(ybuf, [t], hv, mask=m)
                    return 0

                lax.fori_loop(0, CHK // (L * VUNROLL), vec, 0)
                return 0

            lax.fori_loop(0, NCH, chunk, 0)

            pltpu.sync_copy(vi_hbm.at[pl.ds(rb + NCH * CHK, 8)],
                            vtl.at[pl.ds(0, 8)])
            pltpu.sync_copy(h_hbm.at[pl.ds(rb + NCH * CHK, 8)],
                            htl.at[pl.ds(0, 8)])
            v = vtl[...]
            hv = htl[...]
            m = (iota < RTAIL) & (v >= lo) & (v < hi)
            t = jnp.where(m, v - lo, 0)
            plsc.store_scatter(ybuf, [t], hv, mask=m)

        def zero_buf():
            def z(i, _):
                for u in range(VUNROLL):
                    ybuf[pl.ds(i * (L * VUNROLL) + u * L, L)] = zeros
                return 0

            lax.fori_loop(0, YB // (L * VUNROLL), z, 0)

        for pp in range(PPW):
            f = wid * PPW + pp

            zero_buf()
            for sp in range(S):
                scan_row(sp * FOUT + f, 0, HSPLIT)
            for sp in range(S):
                rb = (sp * FOUT + f) * NVP
                pltpu.async_copy(ybuf.at[pl.ds(0, HSPLIT)],
                                 y_hbm.at[pl.ds(rb, HSPLIT)], sem_h)
            for sp in range(S):
                rb = (sp * FOUT + f) * NVP
                pltpu.make_async_copy(ybuf.at[pl.ds(0, HSPLIT)],
                                      y_hbm.at[pl.ds(rb, HSPLIT)],
                                      sem_h).wait()

            zero_buf()
            for sp in range(S):
                scan_row(sp * FOUT + f, HSPLIT, NV)
            for sp in range(S):
                rb = (sp * FOUT + f) * NVP
                pltpu.async_copy(ybuf.at[pl.ds(0, H1W)],
                                 y_hbm.at[pl.ds(rb + HSPLIT, H1W)], sem_h)
            for sp in range(S):
                rb = (sp * FOUT + f) * NVP
                pltpu.make_async_copy(ybuf.at[pl.ds(0, H1W)],
                                      y_hbm.at[pl.ds(rb + HSPLIT, H1W)],
                                      sem_h).wait()

    return k(vi_flat, h_flat)


def _depad_body(s_ref, o_ref):
    o_ref[...] = s_ref[...]


def _depad(y_flat):
    return pl.pallas_call(
        _depad_body,
        grid=(S, 8, NVP // 2688),
        in_specs=[pl.BlockSpec((1, 8, 2688), lambda s, g, j: (s, g, j))],
        out_specs=pl.BlockSpec((1, 8, 2688), lambda s, g, j: (s, g, j)),
        out_shape=jax.ShapeDtypeStruct((S, FOUT, NV), jnp.float32),
    )(y_flat.reshape(S, FOUT, NVP))


def kernel(x, max_pool_indices, up_neigh_indices, down_indices, W, b):
    up_flat = up_neigh_indices.reshape(-1)
    down_pad = jnp.pad(down_indices, (0, DPAD - NR))
    nt_flat = _neigh_gather(up_flat, down_pad)
    b2 = jnp.broadcast_to(b.reshape(FOUT, 1), (FOUT, _NB))
    h, vi = _fc_vi(x, max_pool_indices, nt_flat.reshape(NPL, NRL), W, b2)
    y = _scatter_rows(vi.reshape(-1), h.reshape(-1))
    return _depad(y)


# speedup vs baseline: 1.8724x; 1.8724x over previous
import functools

import jax
import jax.numpy as jnp
from jax import lax
from jax.experimental import pallas as pl
from jax.experimental.pallas import tpu as pltpu
from jax.experimental.pallas import tpu_sc as plsc

S = 2
FIN = 128
FOUT = 64
NR = 40962
NV = 163842
NEI = 7
ROWS = S * FOUT

NC, NS, L = 2, 16, 16
NW = NC * NS

NRL = 41088
NPL = 8
DPAD = 40968

RPW = 1280
RTAIL = NR - NW * RPW
RPA = 1296

NVP = 163968
HSPLIT = 81920
H1W = 82048
YB = 82048
CHK = 8192
NCH = 40960 // CHK
VUNROLL = 8
PPW = FOUT // NW


def _mesh():
    return plsc.VectorSubcoreMesh(core_axis_name="c", subcore_axis_name="s")


def _wid():
    return lax.axis_index("s") * NC + lax.axis_index("c")


def _neigh_gather(up_flat, down_pad):
    @functools.partial(
        pl.kernel,
        mesh=_mesh(),
        out_type=jax.ShapeDtypeStruct((NPL * NRL,), jnp.int32),
        scratch_types=[
            pltpu.VMEM((128,), jnp.int32),
            pltpu.VMEM((NEI * 128,), jnp.int32),
            pltpu.VMEM((NEI * RPA,), jnp.int32),
            pltpu.VMEM((16,), jnp.int32),
            pltpu.SemaphoreType.DMA,
        ],
    )
    def k(up_hbm, down_hbm, out_hbm, didx, gidx, rt_v, dt, sem):
        wid = _wid()
        base = wid * RPW
        for j in range(RPW // 128):
            pltpu.sync_copy(down_hbm.at[pl.ds(base + j * 128, 128)], didx)

            def cv(i, _):
                d16 = didx[pl.ds(i * L, L)]
                for kk in range(NEI):
                    gidx[pl.ds(kk * 128 + i * L, L)] = d16 * NEI + kk
                return 0

            lax.fori_loop(0, 128 // L, cv, 0)
            for kk in range(NEI):
                pltpu.async_copy(up_hbm.at[gidx.at[pl.ds(kk * 128, 128)]],
                                 rt_v.at[pl.ds(kk * RPA + j * 128, 128)],
                                 sem)
            for kk in range(NEI):
                pltpu.make_async_copy(
                    up_hbm.at[gidx.at[pl.ds(kk * 128, 128)]],
                    rt_v.at[pl.ds(kk * RPA + j * 128, 128)], sem).wait()

        @pl.when(wid == NW - 1)
        def _():
            dt[...] = jnp.zeros((L,), jnp.int32)
            pltpu.sync_copy(down_hbm.at[pl.ds(NW * RPW, 8)],
                            dt.at[pl.ds(0, 8)])
            dv = dt[...]
            for kk in range(NEI):
                gidx[pl.ds(kk * 128, L)] = dv * NEI + kk
            for kk in range(NEI):
                pltpu.async_copy(up_hbm.at[gidx.at[pl.ds(kk * 128, L)]],
                                 rt_v.at[pl.ds(kk * RPA + RPW, L)], sem)
            for kk in range(NEI):
                pltpu.make_async_copy(
                    up_hbm.at[gidx.at[pl.ds(kk * 128, L)]],
                    rt_v.at[pl.ds(kk * RPA + RPW, L)], sem).wait()

        for kk in range(NEI):
            @pl.when(wid < NW - 1)
            def _(kk=kk):
                pltpu.sync_copy(rt_v.at[pl.ds(kk * RPA, RPW)],
                                out_hbm.at[pl.ds(kk * NRL + base, RPW)])

            @pl.when(wid == NW - 1)
            def _(kk=kk):
                pltpu.sync_copy(rt_v.at[pl.ds(kk * RPA, 1288)],
                                out_hbm.at[pl.ds(kk * NRL + base, 1288)])

    return k(up_flat, down_pad)


_NB = 4096


def _fc_body(x_ref, mpi_ref, nt_ref, w_ref, b_ref, h_ref, vi_ref):
    h_ref[0] = (
        jnp.dot(w_ref[...], x_ref[0], preferred_element_type=jnp.float32)
        + b_ref[...]
    )
    mpi = mpi_ref[0]
    nt = nt_ref[...]
    acc = jnp.broadcast_to(nt[NEI - 1][None, :], mpi.shape)
    for kk in range(NEI - 2, -1, -1):
        acc = jnp.where(mpi == kk, nt[kk][None, :], acc)
    vi_ref[0] = acc


def _fc_vi(x, mpi, neigh_t, W, b2):
    grid = (S, pl.cdiv(NR, _NB))
    return pl.pallas_call(
        _fc_body,
        grid=grid,
        in_specs=[
            pl.BlockSpec((1, FIN, _NB), lambda s, j: (s, 0, j)),
            pl.BlockSpec((1, FOUT, _NB), lambda s, j: (s, 0, j)),
            pl.BlockSpec((NPL, _NB), lambda s, j: (0, j)),
            pl.BlockSpec((FOUT, FIN), lambda s, j: (0, 0)),
            pl.BlockSpec((FOUT, _NB), lambda s, j: (0, 0)),
        ],
        out_specs=[
            pl.BlockSpec((1, FOUT, _NB), lambda s, j: (s, 0, j)),
            pl.BlockSpec((1, FOUT, _NB), lambda s, j: (s, 0, j)),
        ],
        out_shape=[
            jax.ShapeDtypeStruct((S, FOUT, NRL), jnp.float32),
            jax.ShapeDtypeStruct((S, FOUT, NRL), jnp.int32),
        ],
    )(x, mpi, neigh_t, W, b2)


def _scatter_rows(vi_flat, h_flat):
    @functools.partial(
        pl.kernel,
        mesh=_mesh(),
        out_type=jax.ShapeDtypeStruct((ROWS * NVP,), jnp.float32),
        compiler_params=pltpu.CompilerParams(needs_layout_passes=False),
        scratch_types=[
            pltpu.VMEM((YB,), jnp.float32),
            pltpu.VMEM((2 * CHK,), jnp.int32),
            pltpu.VMEM((2 * CHK,), jnp.float32),
            pltpu.VMEM((16,), jnp.int32),
            pltpu.VMEM((16,), jnp.float32),
            pltpu.SemaphoreType.DMA,
            pltpu.SemaphoreType.DMA,
        ],
    )
    def k(vi_hbm, h_hbm, y_hbm, ybuf, vib, hb, vtl, htl, sem_vi, sem_h):
        wid = _wid()
        iota = lax.iota(jnp.int32, L)
        zeros = jnp.zeros((L,), jnp.float32)

        def scan_row(row, lo, hi):
            rb = row * NRL

            def fire(c, p):
                pltpu.async_copy(vi_hbm.at[pl.ds(rb + c * CHK, CHK)],
                                 vib.at[pl.ds(p * CHK, CHK)], sem_vi)
                pltpu.async_copy(h_hbm.at[pl.ds(rb + c * CHK, CHK)],
                                 hb.at[pl.ds(p * CHK, CHK)], sem_h)

            def wait(c, p):
                pltpu.make_async_copy(vi_hbm.at[pl.ds(rb + c * CHK, CHK)],
                                      vib.at[pl.ds(p * CHK, CHK)],
                                      sem_vi).wait()
                pltpu.make_async_copy(h_hbm.at[pl.ds(rb + c * CHK, CHK)],
                                      hb.at[pl.ds(p * CHK, CHK)],
                                      sem_h).wait()

            fire(0, 0)

            def chunk(c, _):
                p = lax.rem(c, 2)
                wait(c, p)

                @pl.when(c + 1 < NCH)
                def _():
                    fire(c + 1, 1 - p)

                def vec(i, _):
                    for u in range(VUNROLL):
                        off = p * CHK + i * (L * VUNROLL) + u * L
                        v = vib[pl.ds(off, L)]
                        hv = hb[pl.ds(off, L)]
                        m = (v >= lo) & (v < hi)
                        t = jnp.where(m, v - lo, 0)
                        plsc.store_scatter(ybuf, [t], hv, mask=m)
                    return 0

                lax.fori_loop(0, CHK // (L * VUNROLL), vec, 0)
                return 0

            lax.fori_loop(0, NCH, chunk, 0)

            pltpu.sync_copy(vi_hbm.at[pl.ds(rb + NCH * CHK, 8)],
                            vtl.at[pl.ds(0, 8)])
            pltpu.sync_copy(h_hbm.at[pl.ds(rb + NCH * CHK, 8)],
                            htl.at[pl.ds(0, 8)])
            v = vtl[...]
            hv = htl[...]
            m = (iota < RTAIL) & (v >= lo) & (v < hi)
            t = jnp.where(m, v - lo, 0)
            plsc.store_scatter(ybuf, [t], hv, mask=m)

        def zero_buf():
            def z(i, _):
                for u in range(VUNROLL):
                    ybuf[pl.ds(i * (L * VUNROLL) + u * L, L)] = zeros
                return 0

            lax.fori_loop(0, YB // (L * VUNROLL), z, 0)

        for pp in range(PPW):
            f = wid * PPW + pp

            zero_buf()
            for sp in range(S):
                scan_row(sp * FOUT + f, 0, HSPLIT)
            for sp in range(S):
                rb = (sp * FOUT + f) * NVP
                pltpu.async_copy(ybuf.at[pl.ds(0, HSPLIT)],
                                 y_hbm.at[pl.ds(rb, HSPLIT)], sem_h)
            for sp in range(S):
                rb = (sp * FOUT + f) * NVP
                pltpu.make_async_copy(ybuf.at[pl.ds(0, HSPLIT)],
                                      y_hbm.at[pl.ds(rb, HSPLIT)],
                                      sem_h).wait()

            zero_buf()
            for sp in range(S):
                scan_row(sp * FOUT + f, HSPLIT, NV)
            for sp in range(S):
                rb = (sp * FOUT + f) * NVP
                pltpu.async_copy(ybuf.at[pl.ds(0, H1W)],
                                 y_hbm.at[pl.ds(rb + HSPLIT, H1W)], sem_h)
            for sp in range(S):
                rb = (sp * FOUT + f) * NVP
                pltpu.make_async_copy(ybuf.at[pl.ds(0, H1W)],
                                      y_hbm.at[pl.ds(rb + HSPLIT, H1W)],
                                      sem_h).wait()

    return k(vi_flat, h_flat)


def kernel(x, max_pool_indices, up_neigh_indices, down_indices, W, b):
    up_flat = up_neigh_indices.reshape(-1)
    down_pad = jnp.pad(down_indices, (0, DPAD - NR))
    nt_flat = _neigh_gather(up_flat, down_pad)
    b2 = jnp.broadcast_to(b.reshape(FOUT, 1), (FOUT, _NB))
    h, vi = _fc_vi(x, max_pool_indices, nt_flat.reshape(NPL, NRL), W, b2)
    y = _scatter_rows(vi.reshape(-1), h.reshape(-1))
    return y.reshape(S, FOUT, NVP)[:, :, :NV]
